# Initial kernel scaffold; baseline (speedup 1.0000x reference)
import jax
import jax.numpy as jnp
from jax.experimental import pallas as pl


def kernel(x, pos, p_sa1, p_sa2, p_b2, p_b3, p_b4):
    raise NotImplementedError("write your pallas kernel here")


# speedup vs baseline: 2.1407x; 2.1407x over previous
import functools

import jax
import jax.numpy as jnp
from jax.experimental import pallas as pl


def _fps_kernel(pos_ref, ctr_ref, *, n_pts, n_samples):
    px = pos_ref[0, 0:1, :]
    py = pos_ref[0, 1:2, :]
    iota = jax.lax.broadcasted_iota(jnp.int32, (1, n_pts), 1)
    ciota = jax.lax.broadcasted_iota(jnp.int32, (1, n_samples), 1)
    x0 = px[0:1, 0:1]
    y0 = py[0:1, 0:1]
    dmin0 = (px - x0) ** 2 + (py - y0) ** 2
    cx0 = jnp.where(ciota == 0, x0, 0.0)
    cy0 = jnp.where(ciota == 0, y0, 0.0)

    def body(i, carry):
        dmin, cxs, cys = carry
        m = jnp.max(dmin, axis=1, keepdims=True)
        cand = jnp.where(dmin == m, iota, n_pts)
        jsel = jnp.min(cand, axis=1, keepdims=True)
        onehot = iota == jsel
        nx = jnp.sum(jnp.where(onehot, px, 0.0), axis=1, keepdims=True)
        ny = jnp.sum(jnp.where(onehot, py, 0.0), axis=1, keepdims=True)
        dnew = (px - nx) ** 2 + (py - ny) ** 2
        return (jnp.minimum(dmin, dnew),
                jnp.where(ciota == i, nx, cxs),
                jnp.where(ciota == i, ny, cys))

    _, cxs, cys = jax.lax.fori_loop(1, n_samples, body, (dmin0, cx0, cy0))
    ctr_ref[0, 0:1, :] = cxs
    ctr_ref[0, 1:2, :] = cys


def _fps(pos_t, n_samples):
    B, _, N = pos_t.shape
    return pl.pallas_call(
        functools.partial(_fps_kernel, n_pts=N, n_samples=n_samples),
        grid=(B,),
        in_specs=[pl.BlockSpec((1, 2, N), lambda b: (b, 0, 0))],
        out_specs=pl.BlockSpec((1, 2, n_samples), lambda b: (b, 0, 0)),
        out_shape=jax.ShapeDtypeStruct((B, 2, n_samples), jnp.float32),
    )(pos_t)


def _affine_kernel(x_ref, w_ref, b_ref, o_ref):
    o_ref[0, :, :] = (
        jnp.dot(x_ref[0, :, :], w_ref[...], preferred_element_type=jnp.float32)
        + b_ref[...]
    )


def _affine(xp, w, b):
    B, N, D = xp.shape
    Do = w.shape[1]
    return pl.pallas_call(
        _affine_kernel,
        grid=(B,),
        in_specs=[
            pl.BlockSpec((1, N, D), lambda bb: (bb, 0, 0)),
            pl.BlockSpec((D, Do), lambda bb: (0, 0)),
            pl.BlockSpec((1, Do), lambda bb: (0, 0)),
        ],
        out_specs=pl.BlockSpec((1, N, Do), lambda bb: (bb, 0, 0)),
        out_shape=jax.ShapeDtypeStruct((B, N, Do), jnp.float32),
    )(xp, w, b.reshape(1, Do))


def _sa_dual_kernel(a_ref, pos_ref, ctr_ref, wr_ref, w2a_ref, b2a_ref,
                    w2b_ref, b2b_ref, w3b_ref, b3b_ref, oa_ref, ob_ref,
                    *, n_pts, tc, kk, r2):
    A = a_ref[0, :, :]
    px = pos_ref[0, 0:1, :]
    py = pos_ref[0, 1:2, :]
    cx = ctr_ref[0, :, 0:1]
    cy = ctr_ref[0, :, 1:2]
    cp = jnp.dot(ctr_ref[0, :, :], pos_ref[0, :, :],
                 preferred_element_type=jnp.float32)
    d2 = (cx * cx + cy * cy) + (px * px + py * py) - 2.0 * cp
    neg0 = jnp.where(d2 <= r2, -d2, -jnp.inf)
    iota = jax.lax.broadcasted_iota(jnp.int32, (tc, n_pts), 1)
    ctrw = cx * wr_ref[0:1, :] + cy * wr_ref[1:2, :]
    acc_a0 = jnp.full((tc, 128), -jnp.inf, jnp.float32)
    acc_b0 = jnp.full((tc, 256), -jnp.inf, jnp.float32)

    def body(_, carry):
        neg, acc_a, acc_b = carry
        m = jnp.max(neg, axis=1, keepdims=True)
        valid = m > -jnp.inf
        cand = jnp.where(neg == m, iota, n_pts)
        jsel = jnp.min(cand, axis=1, keepdims=True)
        onehot = iota == jsel
        neg = jnp.where(onehot, -jnp.inf, neg)
        g = jnp.dot(onehot.astype(jnp.float32), A,
                    precision=jax.lax.Precision.HIGHEST,
                    preferred_element_type=jnp.float32)
        h1 = jnp.tanh(g - ctrw)
        h2a = jnp.tanh(jnp.dot(h1, w2a_ref[...],
                               preferred_element_type=jnp.float32)
                       + b2a_ref[...])
        acc_a = jnp.where(valid, jnp.maximum(acc_a, h2a), acc_a)
        h2b = jnp.tanh(jnp.dot(h1, w2b_ref[...],
                               preferred_element_type=jnp.float32)
                       + b2b_ref[...])
        h3b = jnp.tanh(jnp.dot(h2b, w3b_ref[...],
                               preferred_element_type=jnp.float32)
                       + b3b_ref[...])
        acc_b = jnp.where(valid, jnp.maximum(acc_b, h3b), acc_b)
        return neg, acc_a, acc_b

    _, acc_a, acc_b = jax.lax.fori_loop(0, kk, body, (neg0, acc_a0, acc_b0))
    oa_ref[0, :, :] = acc_a
    ob_ref[0, :, :] = acc_b


def _sa_dual(A, pos_t, ctr, wr, w2a, b2a, w2b, b2b, w3b, b3b, kk, radius):
    B, N, _ = A.shape
    C = ctr.shape[1]
    TC = 128
    kern = functools.partial(_sa_dual_kernel, n_pts=N, tc=TC, kk=kk,
                             r2=radius * radius)
    return pl.pallas_call(
        kern,
        grid=(B, C // TC),
        in_specs=[
            pl.BlockSpec((1, N, 128), lambda b, t: (b, 0, 0)),
            pl.BlockSpec((1, 2, N), lambda b, t: (b, 0, 0)),
            pl.BlockSpec((1, TC, 2), lambda b, t: (b, t, 0)),
            pl.BlockSpec((2, 128), lambda b, t: (0, 0)),
            pl.BlockSpec((128, 128), lambda b, t: (0, 0)),
            pl.BlockSpec((1, 128), lambda b, t: (0, 0)),
            pl.BlockSpec((128, 128), lambda b, t: (0, 0)),
            pl.BlockSpec((1, 128), lambda b, t: (0, 0)),
            pl.BlockSpec((128, 256), lambda b, t: (0, 0)),
            pl.BlockSpec((1, 256), lambda b, t: (0, 0)),
        ],
        out_specs=[
            pl.BlockSpec((1, TC, 128), lambda b, t: (b, t, 0)),
            pl.BlockSpec((1, TC, 256), lambda b, t: (b, t, 0)),
        ],
        out_shape=[
            jax.ShapeDtypeStruct((B, C, 128), jnp.float32),
            jax.ShapeDtypeStruct((B, C, 256), jnp.float32),
        ],
    )(A, pos_t, ctr, wr, w2a, b2a.reshape(1, -1), w2b, b2b.reshape(1, -1),
      w3b, b3b.reshape(1, -1))


def _sa_single_kernel(a_ref, pos_ref, ctr_ref, wr_ref, o_ref,
                      *, n_pts, tc, kk, r2, dout):
    A = a_ref[0, :, :]
    px = pos_ref[0, 0:1, :]
    py = pos_ref[0, 1:2, :]
    cx = ctr_ref[0, :, 0:1]
    cy = ctr_ref[0, :, 1:2]
    cp = jnp.dot(ctr_ref[0, :, :], pos_ref[0, :, :],
                 preferred_element_type=jnp.float32)
    d2 = (cx * cx + cy * cy) + (px * px + py * py) - 2.0 * cp
    neg0 = jnp.where(d2 <= r2, -d2, -jnp.inf)
    iota = jax.lax.broadcasted_iota(jnp.int32, (tc, n_pts), 1)
    ctrw = cx * wr_ref[0:1, :] + cy * wr_ref[1:2, :]
    acc0 = jnp.full((tc, dout), -jnp.inf, jnp.float32)

    def body(_, carry):
        neg, acc = carry
        m = jnp.max(neg, axis=1, keepdims=True)
        valid = m > -jnp.inf
        cand = jnp.where(neg == m, iota, n_pts)
        jsel = jnp.min(cand, axis=1, keepdims=True)
        onehot = iota == jsel
        neg = jnp.where(onehot, -jnp.inf, neg)
        g = jnp.dot(onehot.astype(jnp.float32), A,
                    precision=jax.lax.Precision.HIGHEST,
                    preferred_element_type=jnp.float32)
        h = jnp.tanh(g - ctrw)
        acc = jnp.where(valid, jnp.maximum(acc, h), acc)
        return neg, acc

    _, acc = jax.lax.fori_loop(0, kk, body, (neg0, acc0))
    o_ref[0, :, :] = acc


def _sa_single(A, pos_t, ctr, wr, kk, radius):
    B, N, Do = A.shape
    C = ctr.shape[1]
    kern = functools.partial(_sa_single_kernel, n_pts=N, tc=C, kk=kk,
                             r2=radius * radius, dout=Do)
    return pl.pallas_call(
        kern,
        grid=(B,),
        in_specs=[
            pl.BlockSpec((1, N, Do), lambda b: (b, 0, 0)),
            pl.BlockSpec((1, 2, N), lambda b: (b, 0, 0)),
            pl.BlockSpec((1, C, 2), lambda b: (b, 0, 0)),
            pl.BlockSpec((2, Do), lambda b: (0, 0)),
        ],
        out_specs=pl.BlockSpec((1, C, Do), lambda b: (b, 0, 0)),
        out_shape=jax.ShapeDtypeStruct((B, C, Do), jnp.float32),
    )(A, pos_t, ctr, wr)


def _make_mlp_max_kernel(n_layers):
    def kern(*refs):
        x_ref = refs[0]
        o_ref = refs[-1]
        h = x_ref[0, :, :]
        for i in range(n_layers):
            w = refs[1 + 2 * i][...]
            b = refs[2 + 2 * i][...]
            h = jnp.tanh(jnp.dot(h, w, preferred_element_type=jnp.float32) + b)
        o_ref[0, :, :] = jnp.max(h, axis=0, keepdims=True)
    return kern


def _mlp_max(xp, params):
    B, M, D = xp.shape
    Do = params[-1][0].shape[1]
    in_specs = [pl.BlockSpec((1, M, D), lambda b: (b, 0, 0))]
    args = [xp]
    for (w, b) in params:
        dw_in, dw_out = w.shape
        in_specs.append(pl.BlockSpec((dw_in, dw_out), lambda b: (0, 0)))
        in_specs.append(pl.BlockSpec((1, dw_out), lambda b: (0, 0)))
        args.append(w)
        args.append(b.reshape(1, dw_out))
    out = pl.pallas_call(
        _make_mlp_max_kernel(len(params)),
        grid=(B,),
        in_specs=in_specs,
        out_specs=pl.BlockSpec((1, 1, Do), lambda b: (b, 0, 0)),
        out_shape=jax.ShapeDtypeStruct((B, 1, Do), jnp.float32),
    )(*args)
    return out[:, 0, :]


def kernel(x, pos, p_sa1, p_sa2, p_b2, p_b3, p_b4):
    B, N, F = x.shape
    n1, n2, K = N // 2, N // 16, 64

    pos_t = jnp.transpose(pos, (0, 2, 1))
    xp = jnp.concatenate([x, pos], axis=-1)

    ctr1_t = _fps(pos_t, n1)
    ctr1 = jnp.transpose(ctr1_t, (0, 2, 1))

    w1a, b1a = p_sa1[0]
    w1b, b1b = p_b2[0]
    wcat = jnp.concatenate([w1a, w1b], axis=1)
    bcat = jnp.concatenate([b1a, b1b], axis=0)
    A1 = _affine(xp, wcat, bcat)
    wr_cat = wcat[F:F + 2, :]

    w2a_pad = jnp.zeros((128, 128), jnp.float32).at[:64, :].set(p_sa1[1][0])
    w2b_pad = jnp.zeros((128, 128), jnp.float32).at[64:, :].set(p_b2[1][0])
    x1a, x2 = _sa_dual(A1, pos_t, ctr1, wr_cat,
                       w2a_pad, p_sa1[1][1], w2b_pad, p_b2[1][1],
                       p_b2[2][0], p_b2[2][1], K, 0.6)

    ctr2_t = _fps(ctr1_t, n2)
    ctr2 = jnp.transpose(ctr2_t, (0, 2, 1))
    xp2 = jnp.concatenate([x1a, ctr1], axis=-1)
    w2, b2 = p_sa2[0]
    A2 = _affine(xp2, w2, b2)
    x1 = _sa_single(A2, ctr1_t, ctr2, w2[128:130, :], K, 0.8)

    x3 = _mlp_max(xp, p_b3)

    feat = jnp.concatenate([x1, x2], axis=1)
    posc = jnp.concatenate([ctr2, ctr1], axis=1)
    xp4 = jnp.concatenate([feat, posc], axis=-1)
    x4 = _mlp_max(xp4, p_b4)

    return jnp.concatenate([x3, x4], axis=-1)


# speedup vs baseline: 2.7882x; 1.3024x over previous
import functools

import jax
import jax.numpy as jnp
from jax.experimental import pallas as pl
from jax.experimental.pallas import tpu as pltpu


def _fps_kernel(pos_ref, ctr_ref, *, n_pts, n_samples):
    px = pos_ref[0, 0:1, :]
    py = pos_ref[0, 1:2, :]
    iota = jax.lax.broadcasted_iota(jnp.int32, (1, n_pts), 1)
    ciota = jax.lax.broadcasted_iota(jnp.int32, (1, n_samples), 1)
    x0 = px[0:1, 0:1]
    y0 = py[0:1, 0:1]
    dmin0 = (px - x0) ** 2 + (py - y0) ** 2
    cx0 = jnp.where(ciota == 0, x0, 0.0)
    cy0 = jnp.where(ciota == 0, y0, 0.0)

    def body(i, carry):
        dmin, cxs, cys = carry
        m = jnp.max(dmin, axis=1, keepdims=True)
        cand = jnp.where(dmin == m, iota, n_pts)
        jsel = jnp.min(cand, axis=1, keepdims=True)
        onehot = iota == jsel
        nx = jnp.sum(jnp.where(onehot, px, 0.0), axis=1, keepdims=True)
        ny = jnp.sum(jnp.where(onehot, py, 0.0), axis=1, keepdims=True)
        dnew = (px - nx) ** 2 + (py - ny) ** 2
        return (jnp.minimum(dmin, dnew),
                jnp.where(ciota == i, nx, cxs),
                jnp.where(ciota == i, ny, cys))

    _, cxs, cys = jax.lax.fori_loop(1, n_samples, body, (dmin0, cx0, cy0))
    ctr_ref[0, 0:1, :] = cxs
    ctr_ref[0, 1:2, :] = cys


def _fps(pos_t, n_samples):
    B, _, N = pos_t.shape
    return pl.pallas_call(
        functools.partial(_fps_kernel, n_pts=N, n_samples=n_samples),
        grid=(B,),
        in_specs=[pl.BlockSpec((1, 2, N), lambda b: (b, 0, 0))],
        out_specs=pl.BlockSpec((1, 2, n_samples), lambda b: (b, 0, 0)),
        out_shape=jax.ShapeDtypeStruct((B, 2, n_samples), jnp.float32),
        compiler_params=pltpu.CompilerParams(
            dimension_semantics=("parallel",)),
    )(pos_t)


def _affine_kernel(x_ref, w_ref, b_ref, o_ref):
    o_ref[0, :, :] = (
        jnp.dot(x_ref[0, :, :], w_ref[...], preferred_element_type=jnp.float32)
        + b_ref[...]
    )


def _affine(xp, w, b):
    B, N, D = xp.shape
    Do = w.shape[1]
    return pl.pallas_call(
        _affine_kernel,
        grid=(B,),
        in_specs=[
            pl.BlockSpec((1, N, D), lambda bb: (bb, 0, 0)),
            pl.BlockSpec((D, Do), lambda bb: (0, 0)),
            pl.BlockSpec((1, Do), lambda bb: (0, 0)),
        ],
        out_specs=pl.BlockSpec((1, N, Do), lambda bb: (bb, 0, 0)),
        out_shape=jax.ShapeDtypeStruct((B, N, Do), jnp.float32),
        compiler_params=pltpu.CompilerParams(
            dimension_semantics=("parallel",)),
    )(xp, w, b.reshape(1, Do))


def _sa_dual_kernel(a_ref, pos_ref, ctr_ref, wr_ref, w2a_ref, b2a_ref,
                    w2b_ref, b2b_ref, w3b_ref, b3b_ref, oa_ref, ob_ref,
                    *, n_pts, tc, kk, r2):
    A = a_ref[0, :, :]
    A_hi = A.astype(jnp.bfloat16)
    A_lo = (A - A_hi.astype(jnp.float32)).astype(jnp.bfloat16)
    px = pos_ref[0, 0:1, :]
    py = pos_ref[0, 1:2, :]
    cx = ctr_ref[0, :, 0:1]
    cy = ctr_ref[0, :, 1:2]
    cp = jnp.dot(ctr_ref[0, :, :], pos_ref[0, :, :],
                 preferred_element_type=jnp.float32)
    d2 = (cx * cx + cy * cy) + (px * px + py * py) - 2.0 * cp
    neg0 = jnp.where(d2 <= r2, -d2, -jnp.inf)
    iota = jax.lax.broadcasted_iota(jnp.int32, (tc, n_pts), 1)
    ctrw = cx * wr_ref[0:1, :] + cy * wr_ref[1:2, :]
    acc_a0 = jnp.full((tc, 128), -jnp.inf, jnp.float32)
    acc_b0 = jnp.full((tc, 256), -jnp.inf, jnp.float32)

    def body(_, carry):
        neg, acc_a, acc_b = carry
        m = jnp.max(neg, axis=1, keepdims=True)
        valid = m > -jnp.inf
        cand = jnp.where(neg == m, iota, n_pts)
        jsel = jnp.min(cand, axis=1, keepdims=True)
        onehot = iota == jsel
        neg = jnp.where(onehot, -jnp.inf, neg)
        oh = onehot.astype(jnp.bfloat16)
        g = (jnp.dot(oh, A_hi, preferred_element_type=jnp.float32)
             + jnp.dot(oh, A_lo, preferred_element_type=jnp.float32))
        h1 = jnp.tanh(g - ctrw)
        h2a = jnp.tanh(jnp.dot(h1, w2a_ref[...],
                               preferred_element_type=jnp.float32)
                       + b2a_ref[...])
        acc_a = jnp.where(valid, jnp.maximum(acc_a, h2a), acc_a)
        h2b = jnp.tanh(jnp.dot(h1, w2b_ref[...],
                               preferred_element_type=jnp.float32)
                       + b2b_ref[...])
        h3b = jnp.tanh(jnp.dot(h2b, w3b_ref[...],
                               preferred_element_type=jnp.float32)
                       + b3b_ref[...])
        acc_b = jnp.where(valid, jnp.maximum(acc_b, h3b), acc_b)
        return neg, acc_a, acc_b

    _, acc_a, acc_b = jax.lax.fori_loop(0, kk, body, (neg0, acc_a0, acc_b0))
    oa_ref[0, :, :] = acc_a
    ob_ref[0, :, :] = acc_b


def _sa_dual(A, pos_t, ctr, wr, w2a, b2a, w2b, b2b, w3b, b3b, kk, radius):
    B, N, _ = A.shape
    C = ctr.shape[1]
    TC = 128
    kern = functools.partial(_sa_dual_kernel, n_pts=N, tc=TC, kk=kk,
                             r2=radius * radius)
    return pl.pallas_call(
        kern,
        grid=(B, C // TC),
        in_specs=[
            pl.BlockSpec((1, N, 128), lambda b, t: (b, 0, 0)),
            pl.BlockSpec((1, 2, N), lambda b, t: (b, 0, 0)),
            pl.BlockSpec((1, TC, 2), lambda b, t: (b, t, 0)),
            pl.BlockSpec((2, 128), lambda b, t: (0, 0)),
            pl.BlockSpec((128, 128), lambda b, t: (0, 0)),
            pl.BlockSpec((1, 128), lambda b, t: (0, 0)),
            pl.BlockSpec((128, 128), lambda b, t: (0, 0)),
            pl.BlockSpec((1, 128), lambda b, t: (0, 0)),
            pl.BlockSpec((128, 256), lambda b, t: (0, 0)),
            pl.BlockSpec((1, 256), lambda b, t: (0, 0)),
        ],
        out_specs=[
            pl.BlockSpec((1, TC, 128), lambda b, t: (b, t, 0)),
            pl.BlockSpec((1, TC, 256), lambda b, t: (b, t, 0)),
        ],
        out_shape=[
            jax.ShapeDtypeStruct((B, C, 128), jnp.float32),
            jax.ShapeDtypeStruct((B, C, 256), jnp.float32),
        ],
        compiler_params=pltpu.CompilerParams(
            dimension_semantics=("parallel", "parallel")),
    )(A, pos_t, ctr, wr, w2a, b2a.reshape(1, -1), w2b, b2b.reshape(1, -1),
      w3b, b3b.reshape(1, -1))


def _sa_single_kernel(a_ref, pos_ref, ctr_ref, wr_ref, o_ref,
                      *, n_pts, tc, kk, r2, dout):
    A = a_ref[0, :, :]
    A_hi = A.astype(jnp.bfloat16)
    A_lo = (A - A_hi.astype(jnp.float32)).astype(jnp.bfloat16)
    px = pos_ref[0, 0:1, :]
    py = pos_ref[0, 1:2, :]
    cx = ctr_ref[0, :, 0:1]
    cy = ctr_ref[0, :, 1:2]
    cp = jnp.dot(ctr_ref[0, :, :], pos_ref[0, :, :],
                 preferred_element_type=jnp.float32)
    d2 = (cx * cx + cy * cy) + (px * px + py * py) - 2.0 * cp
    neg0 = jnp.where(d2 <= r2, -d2, -jnp.inf)
    iota = jax.lax.broadcasted_iota(jnp.int32, (tc, n_pts), 1)
    ctrw = cx * wr_ref[0:1, :] + cy * wr_ref[1:2, :]
    acc0 = jnp.full((tc, dout), -jnp.inf, jnp.float32)

    def body(_, carry):
        neg, acc = carry
        m = jnp.max(neg, axis=1, keepdims=True)
        valid = m > -jnp.inf
        cand = jnp.where(neg == m, iota, n_pts)
        jsel = jnp.min(cand, axis=1, keepdims=True)
        onehot = iota == jsel
        neg = jnp.where(onehot, -jnp.inf, neg)
        oh = onehot.astype(jnp.bfloat16)
        g = (jnp.dot(oh, A_hi, preferred_element_type=jnp.float32)
             + jnp.dot(oh, A_lo, preferred_element_type=jnp.float32))
        h = jnp.tanh(g - ctrw)
        acc = jnp.where(valid, jnp.maximum(acc, h), acc)
        return neg, acc

    _, acc = jax.lax.fori_loop(0, kk, body, (neg0, acc0))
    o_ref[0, :, :] = acc


def _sa_single(A, pos_t, ctr, wr, kk, radius):
    B, N, Do = A.shape
    C = ctr.shape[1]
    kern = functools.partial(_sa_single_kernel, n_pts=N, tc=C, kk=kk,
                             r2=radius * radius, dout=Do)
    return pl.pallas_call(
        kern,
        grid=(B,),
        in_specs=[
            pl.BlockSpec((1, N, Do), lambda b: (b, 0, 0)),
            pl.BlockSpec((1, 2, N), lambda b: (b, 0, 0)),
            pl.BlockSpec((1, C, 2), lambda b: (b, 0, 0)),
            pl.BlockSpec((2, Do), lambda b: (0, 0)),
        ],
        out_specs=pl.BlockSpec((1, C, Do), lambda b: (b, 0, 0)),
        out_shape=jax.ShapeDtypeStruct((B, C, Do), jnp.float32),
        compiler_params=pltpu.CompilerParams(
            dimension_semantics=("parallel",)),
    )(A, pos_t, ctr, wr)


def _make_mlp_max_kernel(n_layers):
    def kern(*refs):
        x_ref = refs[0]
        o_ref = refs[-1]
        h = x_ref[0, :, :]
        for i in range(n_layers):
            w = refs[1 + 2 * i][...]
            b = refs[2 + 2 * i][...]
            h = jnp.tanh(jnp.dot(h, w, preferred_element_type=jnp.float32) + b)
        o_ref[0, :, :] = jnp.max(h, axis=0, keepdims=True)
    return kern


def _mlp_max(xp, params):
    B, M, D = xp.shape
    Do = params[-1][0].shape[1]
    in_specs = [pl.BlockSpec((1, M, D), lambda b: (b, 0, 0))]
    args = [xp]
    for (w, b) in params:
        dw_in, dw_out = w.shape
        in_specs.append(pl.BlockSpec((dw_in, dw_out), lambda b: (0, 0)))
        in_specs.append(pl.BlockSpec((1, dw_out), lambda b: (0, 0)))
        args.append(w)
        args.append(b.reshape(1, dw_out))
    out = pl.pallas_call(
        _make_mlp_max_kernel(len(params)),
        grid=(B,),
        in_specs=in_specs,
        out_specs=pl.BlockSpec((1, 1, Do), lambda b: (b, 0, 0)),
        out_shape=jax.ShapeDtypeStruct((B, 1, Do), jnp.float32),
        compiler_params=pltpu.CompilerParams(
            dimension_semantics=("parallel",)),
    )(*args)
    return out[:, 0, :]


def kernel(x, pos, p_sa1, p_sa2, p_b2, p_b3, p_b4):
    B, N, F = x.shape
    n1, n2, K = N // 2, N // 16, 64

    pos_t = jnp.transpose(pos, (0, 2, 1))
    xp = jnp.concatenate([x, pos], axis=-1)

    ctr1_t = _fps(pos_t, n1)
    ctr1 = jnp.transpose(ctr1_t, (0, 2, 1))

    w1a, b1a = p_sa1[0]
    w1b, b1b = p_b2[0]
    wcat = jnp.concatenate([w1a, w1b], axis=1)
    bcat = jnp.concatenate([b1a, b1b], axis=0)
    A1 = _affine(xp, wcat, bcat)
    wr_cat = wcat[F:F + 2, :]

    w2a_pad = jnp.zeros((128, 128), jnp.float32).at[:64, :].set(p_sa1[1][0])
    w2b_pad = jnp.zeros((128, 128), jnp.float32).at[64:, :].set(p_b2[1][0])
    x1a, x2 = _sa_dual(A1, pos_t, ctr1, wr_cat,
                       w2a_pad, p_sa1[1][1], w2b_pad, p_b2[1][1],
                       p_b2[2][0], p_b2[2][1], K, 0.6)

    ctr2_t = _fps(ctr1_t, n2)
    ctr2 = jnp.transpose(ctr2_t, (0, 2, 1))
    xp2 = jnp.concatenate([x1a, ctr1], axis=-1)
    w2, b2 = p_sa2[0]
    A2 = _affine(xp2, w2, b2)
    x1 = _sa_single(A2, ctr1_t, ctr2, w2[128:130, :], K, 0.8)

    x3 = _mlp_max(xp, p_b3)

    feat = jnp.concatenate([x1, x2], axis=1)
    posc = jnp.concatenate([ctr2, ctr1], axis=1)
    xp4 = jnp.concatenate([feat, posc], axis=-1)
    x4 = _mlp_max(xp4, p_b4)

    return jnp.concatenate([x3, x4], axis=-1)


# speedup vs baseline: 2.8328x; 1.0160x over previous
import functools

import jax
import jax.numpy as jnp
from jax.experimental import pallas as pl
from jax.experimental.pallas import tpu as pltpu


def _red2(op, a):
    return op(op(a, axis=1, keepdims=True), axis=0, keepdims=True)


def _fps_kernel(pos_ref, ctr_ref, *, n_pts, n_samples):
    S, L = pos_ref.shape[2], pos_ref.shape[3]
    CS, CL = ctr_ref.shape[2], ctr_ref.shape[3]
    px = pos_ref[0, 0, :, :]
    py = pos_ref[0, 1, :, :]
    pidx = (jax.lax.broadcasted_iota(jnp.int32, (S, L), 0) * L
            + jax.lax.broadcasted_iota(jnp.int32, (S, L), 1))
    cidx = (jax.lax.broadcasted_iota(jnp.int32, (CS, CL), 0) * CL
            + jax.lax.broadcasted_iota(jnp.int32, (CS, CL), 1))
    x0 = px[0:1, 0:1]
    y0 = py[0:1, 0:1]
    dmin0 = (px - x0) ** 2 + (py - y0) ** 2
    cx0 = jnp.where(cidx == 0, x0, 0.0)
    cy0 = jnp.where(cidx == 0, y0, 0.0)

    def body(i, carry):
        dmin, cxs, cys = carry
        m = _red2(jnp.max, dmin)
        cand = jnp.where(dmin == m, pidx, n_pts)
        jsel = _red2(jnp.min, cand)
        onehot = pidx == jsel
        nx = _red2(jnp.sum, jnp.where(onehot, px, 0.0))
        ny = _red2(jnp.sum, jnp.where(onehot, py, 0.0))
        dnew = (px - nx) ** 2 + (py - ny) ** 2
        return (jnp.minimum(dmin, dnew),
                jnp.where(cidx == i, nx, cxs),
                jnp.where(cidx == i, ny, cys))

    _, cxs, cys = jax.lax.fori_loop(1, n_samples, body, (dmin0, cx0, cy0))
    ctr_ref[0, 0, :, :] = cxs
    ctr_ref[0, 1, :, :] = cys


def _fps(pos_t, n_samples):
    B, _, N = pos_t.shape
    L = 256 if N % 256 == 0 else 128
    S = N // L
    CL = 256 if n_samples % 256 == 0 else 128
    CS = max(n_samples // CL, 1)
    CL = n_samples // CS
    pos_p = pos_t.reshape(B, 2, S, L)
    out = pl.pallas_call(
        functools.partial(_fps_kernel, n_pts=N, n_samples=n_samples),
        grid=(B,),
        in_specs=[pl.BlockSpec((1, 2, S, L), lambda b: (b, 0, 0, 0))],
        out_specs=pl.BlockSpec((1, 2, CS, CL), lambda b: (b, 0, 0, 0)),
        out_shape=jax.ShapeDtypeStruct((B, 2, CS, CL), jnp.float32),
        compiler_params=pltpu.CompilerParams(
            dimension_semantics=("parallel",)),
    )(pos_p)
    return out.reshape(B, 2, n_samples)


def _affine_kernel(x_ref, w_ref, b_ref, o_ref):
    o_ref[0, :, :] = (
        jnp.dot(x_ref[0, :, :], w_ref[...], preferred_element_type=jnp.float32)
        + b_ref[...]
    )


def _affine(xp, w, b):
    B, N, D = xp.shape
    Do = w.shape[1]
    return pl.pallas_call(
        _affine_kernel,
        grid=(B,),
        in_specs=[
            pl.BlockSpec((1, N, D), lambda bb: (bb, 0, 0)),
            pl.BlockSpec((D, Do), lambda bb: (0, 0)),
            pl.BlockSpec((1, Do), lambda bb: (0, 0)),
        ],
        out_specs=pl.BlockSpec((1, N, Do), lambda bb: (bb, 0, 0)),
        out_shape=jax.ShapeDtypeStruct((B, N, Do), jnp.float32),
        compiler_params=pltpu.CompilerParams(
            dimension_semantics=("parallel",)),
    )(xp, w, b.reshape(1, Do))


def _sa_dual_kernel(a_ref, pos_ref, ctr_ref, wr_ref, w2a_ref, b2a_ref,
                    w2b_ref, b2b_ref, w3b_ref, b3b_ref, oa_ref, ob_ref,
                    *, n_pts, tc, kk, r2):
    A = a_ref[0, :, :]
    A_hi = A.astype(jnp.bfloat16)
    A_lo = (A - A_hi.astype(jnp.float32)).astype(jnp.bfloat16)
    px = pos_ref[0, 0:1, :]
    py = pos_ref[0, 1:2, :]
    cx = ctr_ref[0, :, 0:1]
    cy = ctr_ref[0, :, 1:2]
    cp = jnp.dot(ctr_ref[0, :, :], pos_ref[0, :, :],
                 preferred_element_type=jnp.float32)
    d2 = (cx * cx + cy * cy) + (px * px + py * py) - 2.0 * cp
    neg0 = jnp.where(d2 <= r2, -d2, -jnp.inf)
    iota = jax.lax.broadcasted_iota(jnp.int32, (tc, n_pts), 1)
    ctrw = cx * wr_ref[0:1, :] + cy * wr_ref[1:2, :]
    acc_a0 = jnp.full((tc, 128), -jnp.inf, jnp.float32)
    acc_b0 = jnp.full((tc, 256), -jnp.inf, jnp.float32)

    def body(_, carry):
        neg, acc_a, acc_b = carry
        m = jnp.max(neg, axis=1, keepdims=True)
        valid = m > -jnp.inf
        cand = jnp.where(neg == m, iota, n_pts)
        jsel = jnp.min(cand, axis=1, keepdims=True)
        onehot = iota == jsel
        neg = jnp.where(onehot, -jnp.inf, neg)
        oh = onehot.astype(jnp.bfloat16)
        g = (jnp.dot(oh, A_hi, preferred_element_type=jnp.float32)
             + jnp.dot(oh, A_lo, preferred_element_type=jnp.float32))
        h1 = jnp.tanh(g - ctrw)
        h2a = jnp.tanh(jnp.dot(h1, w2a_ref[...],
                               preferred_element_type=jnp.float32)
                       + b2a_ref[...])
        acc_a = jnp.where(valid, jnp.maximum(acc_a, h2a), acc_a)
        h2b = jnp.tanh(jnp.dot(h1, w2b_ref[...],
                               preferred_element_type=jnp.float32)
                       + b2b_ref[...])
        h3b = jnp.tanh(jnp.dot(h2b, w3b_ref[...],
                               preferred_element_type=jnp.float32)
                       + b3b_ref[...])
        acc_b = jnp.where(valid, jnp.maximum(acc_b, h3b), acc_b)
        return neg, acc_a, acc_b

    _, acc_a, acc_b = jax.lax.fori_loop(0, kk, body, (neg0, acc_a0, acc_b0))
    oa_ref[0, :, :] = acc_a
    ob_ref[0, :, :] = acc_b


def _sa_dual(A, pos_t, ctr, wr, w2a, b2a, w2b, b2b, w3b, b3b, kk, radius):
    B, N, _ = A.shape
    C = ctr.shape[1]
    TC = 128
    kern = functools.partial(_sa_dual_kernel, n_pts=N, tc=TC, kk=kk,
                             r2=radius * radius)
    return pl.pallas_call(
        kern,
        grid=(B, C // TC),
        in_specs=[
            pl.BlockSpec((1, N, 128), lambda b, t: (b, 0, 0)),
            pl.BlockSpec((1, 2, N), lambda b, t: (b, 0, 0)),
            pl.BlockSpec((1, TC, 2), lambda b, t: (b, t, 0)),
            pl.BlockSpec((2, 128), lambda b, t: (0, 0)),
            pl.BlockSpec((128, 128), lambda b, t: (0, 0)),
            pl.BlockSpec((1, 128), lambda b, t: (0, 0)),
            pl.BlockSpec((128, 128), lambda b, t: (0, 0)),
            pl.BlockSpec((1, 128), lambda b, t: (0, 0)),
            pl.BlockSpec((128, 256), lambda b, t: (0, 0)),
            pl.BlockSpec((1, 256), lambda b, t: (0, 0)),
        ],
        out_specs=[
            pl.BlockSpec((1, TC, 128), lambda b, t: (b, t, 0)),
            pl.BlockSpec((1, TC, 256), lambda b, t: (b, t, 0)),
        ],
        out_shape=[
            jax.ShapeDtypeStruct((B, C, 128), jnp.float32),
            jax.ShapeDtypeStruct((B, C, 256), jnp.float32),
        ],
        compiler_params=pltpu.CompilerParams(
            dimension_semantics=("parallel", "parallel")),
    )(A, pos_t, ctr, wr, w2a, b2a.reshape(1, -1), w2b, b2b.reshape(1, -1),
      w3b, b3b.reshape(1, -1))


def _sa_single_kernel(a_ref, pos_ref, ctr_ref, wr_ref, o_ref,
                      *, n_pts, tc, kk, r2, dout):
    A = a_ref[0, :, :]
    A_hi = A.astype(jnp.bfloat16)
    A_lo = (A - A_hi.astype(jnp.float32)).astype(jnp.bfloat16)
    px = pos_ref[0, 0:1, :]
    py = pos_ref[0, 1:2, :]
    cx = ctr_ref[0, :, 0:1]
    cy = ctr_ref[0, :, 1:2]
    cp = jnp.dot(ctr_ref[0, :, :], pos_ref[0, :, :],
                 preferred_element_type=jnp.float32)
    d2 = (cx * cx + cy * cy) + (px * px + py * py) - 2.0 * cp
    neg0 = jnp.where(d2 <= r2, -d2, -jnp.inf)
    iota = jax.lax.broadcasted_iota(jnp.int32, (tc, n_pts), 1)
    ctrw = cx * wr_ref[0:1, :] + cy * wr_ref[1:2, :]
    acc0 = jnp.full((tc, dout), -jnp.inf, jnp.float32)

    def body(_, carry):
        neg, acc = carry
        m = jnp.max(neg, axis=1, keepdims=True)
        valid = m > -jnp.inf
        cand = jnp.where(neg == m, iota, n_pts)
        jsel = jnp.min(cand, axis=1, keepdims=True)
        onehot = iota == jsel
        neg = jnp.where(onehot, -jnp.inf, neg)
        oh = onehot.astype(jnp.bfloat16)
        g = (jnp.dot(oh, A_hi, preferred_element_type=jnp.float32)
             + jnp.dot(oh, A_lo, preferred_element_type=jnp.float32))
        h = jnp.tanh(g - ctrw)
        acc = jnp.where(valid, jnp.maximum(acc, h), acc)
        return neg, acc

    _, acc = jax.lax.fori_loop(0, kk, body, (neg0, acc0))
    o_ref[0, :, :] = acc


def _sa_single(A, pos_t, ctr, wr, kk, radius):
    B, N, Do = A.shape
    C = ctr.shape[1]
    kern = functools.partial(_sa_single_kernel, n_pts=N, tc=C, kk=kk,
                             r2=radius * radius, dout=Do)
    return pl.pallas_call(
        kern,
        grid=(B,),
        in_specs=[
            pl.BlockSpec((1, N, Do), lambda b: (b, 0, 0)),
            pl.BlockSpec((1, 2, N), lambda b: (b, 0, 0)),
            pl.BlockSpec((1, C, 2), lambda b: (b, 0, 0)),
            pl.BlockSpec((2, Do), lambda b: (0, 0)),
        ],
        out_specs=pl.BlockSpec((1, C, Do), lambda b: (b, 0, 0)),
        out_shape=jax.ShapeDtypeStruct((B, C, Do), jnp.float32),
        compiler_params=pltpu.CompilerParams(
            dimension_semantics=("parallel",)),
    )(A, pos_t, ctr, wr)


def _make_mlp_max_kernel(n_layers):
    def kern(*refs):
        x_ref = refs[0]
        o_ref = refs[-1]
        h = x_ref[0, :, :]
        for i in range(n_layers):
            w = refs[1 + 2 * i][...]
            b = refs[2 + 2 * i][...]
            h = jnp.tanh(jnp.dot(h, w, preferred_element_type=jnp.float32) + b)
        o_ref[0, :, :] = jnp.max(h, axis=0, keepdims=True)
    return kern


def _mlp_max(xp, params):
    B, M, D = xp.shape
    Do = params[-1][0].shape[1]
    in_specs = [pl.BlockSpec((1, M, D), lambda b: (b, 0, 0))]
    args = [xp]
    for (w, b) in params:
        dw_in, dw_out = w.shape
        in_specs.append(pl.BlockSpec((dw_in, dw_out), lambda b: (0, 0)))
        in_specs.append(pl.BlockSpec((1, dw_out), lambda b: (0, 0)))
        args.append(w)
        args.append(b.reshape(1, dw_out))
    out = pl.pallas_call(
        _make_mlp_max_kernel(len(params)),
        grid=(B,),
        in_specs=in_specs,
        out_specs=pl.BlockSpec((1, 1, Do), lambda b: (b, 0, 0)),
        out_shape=jax.ShapeDtypeStruct((B, 1, Do), jnp.float32),
        compiler_params=pltpu.CompilerParams(
            dimension_semantics=("parallel",)),
    )(*args)
    return out[:, 0, :]


def kernel(x, pos, p_sa1, p_sa2, p_b2, p_b3, p_b4):
    B, N, F = x.shape
    n1, n2, K = N // 2, N // 16, 64

    pos_t = jnp.transpose(pos, (0, 2, 1))
    xp = jnp.concatenate([x, pos], axis=-1)

    ctr1_t = _fps(pos_t, n1)
    ctr1 = jnp.transpose(ctr1_t, (0, 2, 1))

    w1a, b1a = p_sa1[0]
    w1b, b1b = p_b2[0]
    wcat = jnp.concatenate([w1a, w1b], axis=1)
    bcat = jnp.concatenate([b1a, b1b], axis=0)
    A1 = _affine(xp, wcat, bcat)
    wr_cat = wcat[F:F + 2, :]

    w2a_pad = jnp.zeros((128, 128), jnp.float32).at[:64, :].set(p_sa1[1][0])
    w2b_pad = jnp.zeros((128, 128), jnp.float32).at[64:, :].set(p_b2[1][0])
    x1a, x2 = _sa_dual(A1, pos_t, ctr1, wr_cat,
                       w2a_pad, p_sa1[1][1], w2b_pad, p_b2[1][1],
                       p_b2[2][0], p_b2[2][1], K, 0.6)

    ctr2_t = _fps(ctr1_t, n2)
    ctr2 = jnp.transpose(ctr2_t, (0, 2, 1))
    xp2 = jnp.concatenate([x1a, ctr1], axis=-1)
    w2, b2 = p_sa2[0]
    A2 = _affine(xp2, w2, b2)
    x1 = _sa_single(A2, ctr1_t, ctr2, w2[128:130, :], K, 0.8)

    x3 = _mlp_max(xp, p_b3)

    feat = jnp.concatenate([x1, x2], axis=1)
    posc = jnp.concatenate([ctr2, ctr1], axis=1)
    xp4 = jnp.concatenate([feat, posc], axis=-1)
    x4 = _mlp_max(xp4, p_b4)

    return jnp.concatenate([x3, x4], axis=-1)


# speedup vs baseline: 3.4808x; 1.2288x over previous
import functools

import jax
import jax.numpy as jnp
from jax.experimental import pallas as pl
from jax.experimental.pallas import tpu as pltpu


def _red2(op, a):
    return op(op(a, axis=1, keepdims=True), axis=0, keepdims=True)


def _fps_kernel(pos_ref, ctr_ref, *, n_pts, n_samples):
    S, L = pos_ref.shape[2], pos_ref.shape[3]
    CS, CL = ctr_ref.shape[2], ctr_ref.shape[3]
    px = pos_ref[0, 0, :, :]
    py = pos_ref[0, 1, :, :]
    pidx = (jax.lax.broadcasted_iota(jnp.int32, (S, L), 0) * L
            + jax.lax.broadcasted_iota(jnp.int32, (S, L), 1))
    cidx = (jax.lax.broadcasted_iota(jnp.int32, (CS, CL), 0) * CL
            + jax.lax.broadcasted_iota(jnp.int32, (CS, CL), 1))
    x0 = px[0:1, 0:1]
    y0 = py[0:1, 0:1]
    dmin0 = (px - x0) ** 2 + (py - y0) ** 2
    cx0 = jnp.where(cidx == 0, x0, 0.0)
    cy0 = jnp.where(cidx == 0, y0, 0.0)

    def body(i, carry):
        dmin, cxs, cys = carry
        m = _red2(jnp.max, dmin)
        cand = jnp.where(dmin == m, pidx, n_pts)
        jsel = _red2(jnp.min, cand)
        onehot = pidx == jsel
        nx = _red2(jnp.sum, jnp.where(onehot, px, 0.0))
        ny = _red2(jnp.sum, jnp.where(onehot, py, 0.0))
        dnew = (px - nx) ** 2 + (py - ny) ** 2
        return (jnp.minimum(dmin, dnew),
                jnp.where(cidx == i, nx, cxs),
                jnp.where(cidx == i, ny, cys))

    _, cxs, cys = jax.lax.fori_loop(1, n_samples, body, (dmin0, cx0, cy0))
    ctr_ref[0, 0, :, :] = cxs
    ctr_ref[0, 1, :, :] = cys


def _fps(pos_t, n_samples):
    B, _, N = pos_t.shape
    L = 256 if N % 256 == 0 else 128
    S = N // L
    CL = 256 if n_samples % 256 == 0 else 128
    CS = max(n_samples // CL, 1)
    CL = n_samples // CS
    pos_p = pos_t.reshape(B, 2, S, L)
    out = pl.pallas_call(
        functools.partial(_fps_kernel, n_pts=N, n_samples=n_samples),
        grid=(B,),
        in_specs=[pl.BlockSpec((1, 2, S, L), lambda b: (b, 0, 0, 0))],
        out_specs=pl.BlockSpec((1, 2, CS, CL), lambda b: (b, 0, 0, 0)),
        out_shape=jax.ShapeDtypeStruct((B, 2, CS, CL), jnp.float32),
        compiler_params=pltpu.CompilerParams(
            dimension_semantics=("parallel",)),
    )(pos_p)
    return out.reshape(B, 2, n_samples)


def _affine_kernel(x_ref, w_ref, b_ref, o_ref):
    o_ref[0, :, :] = (
        jnp.dot(x_ref[0, :, :], w_ref[...], preferred_element_type=jnp.float32)
        + b_ref[...]
    )


def _affine(xp, w, b):
    B, N, D = xp.shape
    Do = w.shape[1]
    return pl.pallas_call(
        _affine_kernel,
        grid=(B,),
        in_specs=[
            pl.BlockSpec((1, N, D), lambda bb: (bb, 0, 0)),
            pl.BlockSpec((D, Do), lambda bb: (0, 0)),
            pl.BlockSpec((1, Do), lambda bb: (0, 0)),
        ],
        out_specs=pl.BlockSpec((1, N, Do), lambda bb: (bb, 0, 0)),
        out_shape=jax.ShapeDtypeStruct((B, N, Do), jnp.float32),
        compiler_params=pltpu.CompilerParams(
            dimension_semantics=("parallel",)),
    )(xp, w, b.reshape(1, Do))


def _sa_dual_kernel(a_ref, pos_ref, ctr_ref, wr_ref, w2a_ref, b2a_ref,
                    w2b_ref, b2b_ref, w3b_ref, b3b_ref, oa_ref, ob_ref,
                    *, n_pts, tc, kk, r2):
    A = a_ref[0, :, :]
    A_hi = A.astype(jnp.bfloat16)
    A_lo = (A - A_hi.astype(jnp.float32)).astype(jnp.bfloat16)
    AHL = jnp.concatenate([A_hi, A_lo], axis=1)
    px = pos_ref[0, 0:1, :]
    py = pos_ref[0, 1:2, :]
    cx = ctr_ref[0, :, 0:1]
    cy = ctr_ref[0, :, 1:2]
    cp = jnp.dot(ctr_ref[0, :, :], pos_ref[0, :, :],
                 preferred_element_type=jnp.float32)
    d2 = (cx * cx + cy * cy) + (px * px + py * py) - 2.0 * cp
    neg0 = jnp.where(d2 <= r2, -d2, -jnp.inf)
    iota = jax.lax.broadcasted_iota(jnp.int32, (tc, n_pts), 1)
    ctrw = cx * wr_ref[0:1, :] + cy * wr_ref[1:2, :]
    acc_a0 = jnp.full((tc, 128), -jnp.inf, jnp.float32)
    acc_b0 = jnp.full((tc, 256), -jnp.inf, jnp.float32)

    def body(_, carry):
        neg, acc_a, acc_b = carry
        m = jnp.max(neg, axis=1, keepdims=True)
        valid = m > -jnp.inf
        cand = jnp.where(neg == m, iota, n_pts)
        jsel = jnp.min(cand, axis=1, keepdims=True)
        onehot = iota == jsel
        neg = jnp.where(onehot, -jnp.inf, neg)
        oh = onehot.astype(jnp.bfloat16)
        g2 = jnp.dot(oh, AHL, preferred_element_type=jnp.float32)
        g = g2[:, 0:128] + g2[:, 128:256]
        h1 = jnp.tanh(g - ctrw)
        h2a = jnp.tanh(jnp.dot(h1, w2a_ref[...],
                               preferred_element_type=jnp.float32)
                       + b2a_ref[...])
        acc_a = jnp.where(valid, jnp.maximum(acc_a, h2a), acc_a)
        h2b = jnp.tanh(jnp.dot(h1, w2b_ref[...],
                               preferred_element_type=jnp.float32)
                       + b2b_ref[...])
        h3b = jnp.tanh(jnp.dot(h2b, w3b_ref[...],
                               preferred_element_type=jnp.float32)
                       + b3b_ref[...])
        acc_b = jnp.where(valid, jnp.maximum(acc_b, h3b), acc_b)
        return neg, acc_a, acc_b

    _, acc_a, acc_b = jax.lax.fori_loop(0, kk, body, (neg0, acc_a0, acc_b0))
    oa_ref[0, :, :] = acc_a
    ob_ref[0, :, :] = acc_b


def _sa_dual(A, pos_t, ctr, wr, w2a, b2a, w2b, b2b, w3b, b3b, kk, radius):
    B, N, _ = A.shape
    C = ctr.shape[1]
    TC = 256
    kern = functools.partial(_sa_dual_kernel, n_pts=N, tc=TC, kk=kk,
                             r2=radius * radius)
    return pl.pallas_call(
        kern,
        grid=(B, C // TC),
        in_specs=[
            pl.BlockSpec((1, N, 128), lambda b, t: (b, 0, 0)),
            pl.BlockSpec((1, 2, N), lambda b, t: (b, 0, 0)),
            pl.BlockSpec((1, TC, 2), lambda b, t: (b, t, 0)),
            pl.BlockSpec((2, 128), lambda b, t: (0, 0)),
            pl.BlockSpec((128, 128), lambda b, t: (0, 0)),
            pl.BlockSpec((1, 128), lambda b, t: (0, 0)),
            pl.BlockSpec((128, 128), lambda b, t: (0, 0)),
            pl.BlockSpec((1, 128), lambda b, t: (0, 0)),
            pl.BlockSpec((128, 256), lambda b, t: (0, 0)),
            pl.BlockSpec((1, 256), lambda b, t: (0, 0)),
        ],
        out_specs=[
            pl.BlockSpec((1, TC, 128), lambda b, t: (b, t, 0)),
            pl.BlockSpec((1, TC, 256), lambda b, t: (b, t, 0)),
        ],
        out_shape=[
            jax.ShapeDtypeStruct((B, C, 128), jnp.float32),
            jax.ShapeDtypeStruct((B, C, 256), jnp.float32),
        ],
        compiler_params=pltpu.CompilerParams(
            dimension_semantics=("parallel", "parallel")),
    )(A, pos_t, ctr, wr, w2a, b2a.reshape(1, -1), w2b, b2b.reshape(1, -1),
      w3b, b3b.reshape(1, -1))


def _sa_single_kernel(a_ref, pos_ref, ctr_ref, wr_ref, o_ref,
                      *, n_pts, tc, kk, r2, dout):
    A = a_ref[0, :, :]
    A_hi = A.astype(jnp.bfloat16)
    A_lo = (A - A_hi.astype(jnp.float32)).astype(jnp.bfloat16)
    AHL = jnp.concatenate([A_hi, A_lo], axis=1)
    px = pos_ref[0, 0:1, :]
    py = pos_ref[0, 1:2, :]
    cx = ctr_ref[0, :, 0:1]
    cy = ctr_ref[0, :, 1:2]
    cp = jnp.dot(ctr_ref[0, :, :], pos_ref[0, :, :],
                 preferred_element_type=jnp.float32)
    d2 = (cx * cx + cy * cy) + (px * px + py * py) - 2.0 * cp
    neg0 = jnp.where(d2 <= r2, -d2, -jnp.inf)
    iota = jax.lax.broadcasted_iota(jnp.int32, (tc, n_pts), 1)
    ctrw = cx * wr_ref[0:1, :] + cy * wr_ref[1:2, :]
    acc0 = jnp.full((tc, dout), -jnp.inf, jnp.float32)

    def body(_, carry):
        neg, acc = carry
        m = jnp.max(neg, axis=1, keepdims=True)
        valid = m > -jnp.inf
        cand = jnp.where(neg == m, iota, n_pts)
        jsel = jnp.min(cand, axis=1, keepdims=True)
        onehot = iota == jsel
        neg = jnp.where(onehot, -jnp.inf, neg)
        oh = onehot.astype(jnp.bfloat16)
        g2 = jnp.dot(oh, AHL, preferred_element_type=jnp.float32)
        g = g2[:, 0:dout] + g2[:, dout:2 * dout]
        h = jnp.tanh(g - ctrw)
        acc = jnp.where(valid, jnp.maximum(acc, h), acc)
        return neg, acc

    _, acc = jax.lax.fori_loop(0, kk, body, (neg0, acc0))
    o_ref[0, :, :] = acc


def _sa_single(A, pos_t, ctr, wr, kk, radius):
    B, N, Do = A.shape
    C = ctr.shape[1]
    kern = functools.partial(_sa_single_kernel, n_pts=N, tc=C, kk=kk,
                             r2=radius * radius, dout=Do)
    return pl.pallas_call(
        kern,
        grid=(B,),
        in_specs=[
            pl.BlockSpec((1, N, Do), lambda b: (b, 0, 0)),
            pl.BlockSpec((1, 2, N), lambda b: (b, 0, 0)),
            pl.BlockSpec((1, C, 2), lambda b: (b, 0, 0)),
            pl.BlockSpec((2, Do), lambda b: (0, 0)),
        ],
        out_specs=pl.BlockSpec((1, C, Do), lambda b: (b, 0, 0)),
        out_shape=jax.ShapeDtypeStruct((B, C, Do), jnp.float32),
        compiler_params=pltpu.CompilerParams(
            dimension_semantics=("parallel",)),
    )(A, pos_t, ctr, wr)


def _make_mlp_max_kernel(n_layers):
    def kern(*refs):
        x_ref = refs[0]
        o_ref = refs[-1]
        h = x_ref[0, :, :]
        for i in range(n_layers):
            w = refs[1 + 2 * i][...]
            b = refs[2 + 2 * i][...]
            h = jnp.tanh(jnp.dot(h, w, preferred_element_type=jnp.float32) + b)
        o_ref[0, :, :] = jnp.max(h, axis=0, keepdims=True)
    return kern


def _mlp_max(xp, params):
    B, M, D = xp.shape
    Do = params[-1][0].shape[1]
    in_specs = [pl.BlockSpec((1, M, D), lambda b: (b, 0, 0))]
    args = [xp]
    for (w, b) in params:
        dw_in, dw_out = w.shape
        in_specs.append(pl.BlockSpec((dw_in, dw_out), lambda b: (0, 0)))
        in_specs.append(pl.BlockSpec((1, dw_out), lambda b: (0, 0)))
        args.append(w)
        args.append(b.reshape(1, dw_out))
    out = pl.pallas_call(
        _make_mlp_max_kernel(len(params)),
        grid=(B,),
        in_specs=in_specs,
        out_specs=pl.BlockSpec((1, 1, Do), lambda b: (b, 0, 0)),
        out_shape=jax.ShapeDtypeStruct((B, 1, Do), jnp.float32),
        compiler_params=pltpu.CompilerParams(
            dimension_semantics=("parallel",)),
    )(*args)
    return out[:, 0, :]


def kernel(x, pos, p_sa1, p_sa2, p_b2, p_b3, p_b4):
    B, N, F = x.shape
    n1, n2, K = N // 2, N // 16, 64

    pos_t = jnp.transpose(pos, (0, 2, 1))
    xp = jnp.concatenate([x, pos], axis=-1)

    ctr1_t = _fps(pos_t, n1)
    ctr1 = jnp.transpose(ctr1_t, (0, 2, 1))

    w1a, b1a = p_sa1[0]
    w1b, b1b = p_b2[0]
    wcat = jnp.concatenate([w1a, w1b], axis=1)
    bcat = jnp.concatenate([b1a, b1b], axis=0)
    A1 = _affine(xp, wcat, bcat)
    wr_cat = wcat[F:F + 2, :]

    w2a_pad = jnp.zeros((128, 128), jnp.float32).at[:64, :].set(p_sa1[1][0])
    w2b_pad = jnp.zeros((128, 128), jnp.float32).at[64:, :].set(p_b2[1][0])
    x1a, x2 = _sa_dual(A1, pos_t, ctr1, wr_cat,
                       w2a_pad, p_sa1[1][1], w2b_pad, p_b2[1][1],
                       p_b2[2][0], p_b2[2][1], K, 0.6)

    ctr2_t = _fps(ctr1_t, n2)
    ctr2 = jnp.transpose(ctr2_t, (0, 2, 1))
    xp2 = jnp.concatenate([x1a, ctr1], axis=-1)
    w2, b2 = p_sa2[0]
    A2 = _affine(xp2, w2, b2)
    x1 = _sa_single(A2, ctr1_t, ctr2, w2[128:130, :], K, 0.8)

    x3 = _mlp_max(xp, p_b3)

    feat = jnp.concatenate([x1, x2], axis=1)
    posc = jnp.concatenate([ctr2, ctr1], axis=1)
    xp4 = jnp.concatenate([feat, posc], axis=-1)
    x4 = _mlp_max(xp4, p_b4)

    return jnp.concatenate([x3, x4], axis=-1)


# speedup vs baseline: 3.7555x; 1.0789x over previous
import functools

import jax
import jax.numpy as jnp
from jax.experimental import pallas as pl
from jax.experimental.pallas import tpu as pltpu


def _red2(op, a):
    return op(op(a, axis=1, keepdims=True), axis=0, keepdims=True)


def _fps_kernel(pos_ref, ctr_ref, *, n_pts, n_samples):
    S, L = pos_ref.shape[2], pos_ref.shape[3]
    CS, CL = ctr_ref.shape[2], ctr_ref.shape[3]
    px = pos_ref[0, 0, :, :]
    py = pos_ref[0, 1, :, :]
    pidx = (jax.lax.broadcasted_iota(jnp.int32, (S, L), 0) * L
            + jax.lax.broadcasted_iota(jnp.int32, (S, L), 1))
    cidx = (jax.lax.broadcasted_iota(jnp.int32, (CS, CL), 0) * CL
            + jax.lax.broadcasted_iota(jnp.int32, (CS, CL), 1))
    x0 = px[0:1, 0:1]
    y0 = py[0:1, 0:1]
    dmin0 = (px - x0) ** 2 + (py - y0) ** 2
    cx0 = jnp.where(cidx == 0, x0, 0.0)
    cy0 = jnp.where(cidx == 0, y0, 0.0)

    def body(i, carry):
        dmin, cxs, cys = carry
        m = _red2(jnp.max, dmin)
        cand = jnp.where(dmin == m, pidx, n_pts)
        jsel = _red2(jnp.min, cand)
        onehot = pidx == jsel
        nx = _red2(jnp.sum, jnp.where(onehot, px, 0.0))
        ny = _red2(jnp.sum, jnp.where(onehot, py, 0.0))
        dnew = (px - nx) ** 2 + (py - ny) ** 2
        return (jnp.minimum(dmin, dnew),
                jnp.where(cidx == i, nx, cxs),
                jnp.where(cidx == i, ny, cys))

    _, cxs, cys = jax.lax.fori_loop(1, n_samples, body, (dmin0, cx0, cy0))
    ctr_ref[0, 0, :, :] = cxs
    ctr_ref[0, 1, :, :] = cys


def _fps(pos_t, n_samples):
    B, _, N = pos_t.shape
    L = 256 if N % 256 == 0 else 128
    S = N // L
    CL = 256 if n_samples % 256 == 0 else 128
    CS = max(n_samples // CL, 1)
    CL = n_samples // CS
    pos_p = pos_t.reshape(B, 2, S, L)
    out = pl.pallas_call(
        functools.partial(_fps_kernel, n_pts=N, n_samples=n_samples),
        grid=(B,),
        in_specs=[pl.BlockSpec((1, 2, S, L), lambda b: (b, 0, 0, 0))],
        out_specs=pl.BlockSpec((1, 2, CS, CL), lambda b: (b, 0, 0, 0)),
        out_shape=jax.ShapeDtypeStruct((B, 2, CS, CL), jnp.float32),
        compiler_params=pltpu.CompilerParams(
            dimension_semantics=("parallel",)),
    )(pos_p)
    return out.reshape(B, 2, n_samples)


def _affine_kernel(x_ref, w_ref, b_ref, o_ref):
    o_ref[0, :, :] = (
        jnp.dot(x_ref[0, :, :], w_ref[...], preferred_element_type=jnp.float32)
        + b_ref[...]
    )


def _affine(xp, w, b):
    B, N, D = xp.shape
    Do = w.shape[1]
    return pl.pallas_call(
        _affine_kernel,
        grid=(B,),
        in_specs=[
            pl.BlockSpec((1, N, D), lambda bb: (bb, 0, 0)),
            pl.BlockSpec((D, Do), lambda bb: (0, 0)),
            pl.BlockSpec((1, Do), lambda bb: (0, 0)),
        ],
        out_specs=pl.BlockSpec((1, N, Do), lambda bb: (bb, 0, 0)),
        out_shape=jax.ShapeDtypeStruct((B, N, Do), jnp.float32),
        compiler_params=pltpu.CompilerParams(
            dimension_semantics=("parallel",)),
    )(xp, w, b.reshape(1, Do))


def _sa_dual_kernel(a_ref, pos_ref, ctr_ref, wr_ref, w2a_ref, b2a_ref,
                    w2b_ref, b2b_ref, w3b_ref, b3b_ref, oa_ref, ob_ref,
                    *, n_pts, tc, kk, r2):
    A = a_ref[0, :, :]
    A_hi = A.astype(jnp.bfloat16)
    A_lo = (A - A_hi.astype(jnp.float32)).astype(jnp.bfloat16)
    AHL = jnp.concatenate([A_hi, A_lo], axis=1)
    px = pos_ref[0, 0:1, :]
    py = pos_ref[0, 1:2, :]
    cx = ctr_ref[0, :, 0:1]
    cy = ctr_ref[0, :, 1:2]
    cp = jnp.dot(ctr_ref[0, :, :], pos_ref[0, :, :],
                 preferred_element_type=jnp.float32)
    d2 = (cx * cx + cy * cy) + (px * px + py * py) - 2.0 * cp
    neg0 = jnp.where(d2 <= r2, -d2, -jnp.inf)
    iota = jax.lax.broadcasted_iota(jnp.int32, (tc, n_pts), 1)
    ctrw = cx * wr_ref[0:1, :] + cy * wr_ref[1:2, :]
    acc_a0 = jnp.full((tc, 128), -jnp.inf, jnp.float32)
    acc_b0 = jnp.full((tc, 256), -jnp.inf, jnp.float32)

    def body(_, carry):
        neg, acc_a, acc_b = carry
        m = jnp.max(neg, axis=1, keepdims=True)
        valid = m > -jnp.inf
        cand = jnp.where(neg == m, iota, n_pts)
        jsel = jnp.min(cand, axis=1, keepdims=True)
        onehot = iota == jsel
        neg = jnp.where(onehot, -jnp.inf, neg)
        oh = onehot.astype(jnp.bfloat16)
        g2 = jnp.dot(oh, AHL, preferred_element_type=jnp.float32)
        g = g2[:, 0:128] + g2[:, 128:256]
        h1 = jnp.tanh(g - ctrw)
        h2a = jnp.tanh(jnp.dot(h1, w2a_ref[...],
                               preferred_element_type=jnp.float32)
                       + b2a_ref[...])
        acc_a = jnp.where(valid, jnp.maximum(acc_a, h2a), acc_a)
        h2b = jnp.tanh(jnp.dot(h1, w2b_ref[...],
                               preferred_element_type=jnp.float32)
                       + b2b_ref[...])
        h3b = jnp.tanh(jnp.dot(h2b, w3b_ref[...],
                               preferred_element_type=jnp.float32)
                       + b3b_ref[...])
        acc_b = jnp.where(valid, jnp.maximum(acc_b, h3b), acc_b)
        return neg, acc_a, acc_b

    _, acc_a, acc_b = jax.lax.fori_loop(0, kk, body, (neg0, acc_a0, acc_b0))
    oa_ref[0, :, :] = acc_a
    ob_ref[0, :, :] = acc_b


def _sa_dual(A, pos_t, ctr, wr, w2a, b2a, w2b, b2b, w3b, b3b, kk, radius):
    B, N, _ = A.shape
    C = ctr.shape[1]
    TC = 512
    kern = functools.partial(_sa_dual_kernel, n_pts=N, tc=TC, kk=kk,
                             r2=radius * radius)
    return pl.pallas_call(
        kern,
        grid=(B, C // TC),
        in_specs=[
            pl.BlockSpec((1, N, 128), lambda b, t: (b, 0, 0)),
            pl.BlockSpec((1, 2, N), lambda b, t: (b, 0, 0)),
            pl.BlockSpec((1, TC, 2), lambda b, t: (b, t, 0)),
            pl.BlockSpec((2, 128), lambda b, t: (0, 0)),
            pl.BlockSpec((128, 128), lambda b, t: (0, 0)),
            pl.BlockSpec((1, 128), lambda b, t: (0, 0)),
            pl.BlockSpec((128, 128), lambda b, t: (0, 0)),
            pl.BlockSpec((1, 128), lambda b, t: (0, 0)),
            pl.BlockSpec((128, 256), lambda b, t: (0, 0)),
            pl.BlockSpec((1, 256), lambda b, t: (0, 0)),
        ],
        out_specs=[
            pl.BlockSpec((1, TC, 128), lambda b, t: (b, t, 0)),
            pl.BlockSpec((1, TC, 256), lambda b, t: (b, t, 0)),
        ],
        out_shape=[
            jax.ShapeDtypeStruct((B, C, 128), jnp.float32),
            jax.ShapeDtypeStruct((B, C, 256), jnp.float32),
        ],
        compiler_params=pltpu.CompilerParams(
            dimension_semantics=("parallel", "parallel")),
    )(A, pos_t, ctr, wr, w2a, b2a.reshape(1, -1), w2b, b2b.reshape(1, -1),
      w3b, b3b.reshape(1, -1))


def _sa_single_kernel(a_ref, pos_ref, ctr_ref, wr_ref, o_ref,
                      *, n_pts, tc, kk, r2, dout):
    A = a_ref[0, :, :]
    A_hi = A.astype(jnp.bfloat16)
    A_lo = (A - A_hi.astype(jnp.float32)).astype(jnp.bfloat16)
    AHL = jnp.concatenate([A_hi, A_lo], axis=1)
    px = pos_ref[0, 0:1, :]
    py = pos_ref[0, 1:2, :]
    cx = ctr_ref[0, :, 0:1]
    cy = ctr_ref[0, :, 1:2]
    cp = jnp.dot(ctr_ref[0, :, :], pos_ref[0, :, :],
                 preferred_element_type=jnp.float32)
    d2 = (cx * cx + cy * cy) + (px * px + py * py) - 2.0 * cp
    neg0 = jnp.where(d2 <= r2, -d2, -jnp.inf)
    iota = jax.lax.broadcasted_iota(jnp.int32, (tc, n_pts), 1)
    ctrw = cx * wr_ref[0:1, :] + cy * wr_ref[1:2, :]
    acc0 = jnp.full((tc, dout), -jnp.inf, jnp.float32)

    def body(_, carry):
        neg, acc = carry
        m = jnp.max(neg, axis=1, keepdims=True)
        valid = m > -jnp.inf
        cand = jnp.where(neg == m, iota, n_pts)
        jsel = jnp.min(cand, axis=1, keepdims=True)
        onehot = iota == jsel
        neg = jnp.where(onehot, -jnp.inf, neg)
        oh = onehot.astype(jnp.bfloat16)
        g2 = jnp.dot(oh, AHL, preferred_element_type=jnp.float32)
        g = g2[:, 0:dout] + g2[:, dout:2 * dout]
        h = jnp.tanh(g - ctrw)
        acc = jnp.where(valid, jnp.maximum(acc, h), acc)
        return neg, acc

    _, acc = jax.lax.fori_loop(0, kk, body, (neg0, acc0))
    o_ref[0, :, :] = acc


def _sa_single(A, pos_t, ctr, wr, kk, radius):
    B, N, Do = A.shape
    C = ctr.shape[1]
    kern = functools.partial(_sa_single_kernel, n_pts=N, tc=C, kk=kk,
                             r2=radius * radius, dout=Do)
    return pl.pallas_call(
        kern,
        grid=(B,),
        in_specs=[
            pl.BlockSpec((1, N, Do), lambda b: (b, 0, 0)),
            pl.BlockSpec((1, 2, N), lambda b: (b, 0, 0)),
            pl.BlockSpec((1, C, 2), lambda b: (b, 0, 0)),
            pl.BlockSpec((2, Do), lambda b: (0, 0)),
        ],
        out_specs=pl.BlockSpec((1, C, Do), lambda b: (b, 0, 0)),
        out_shape=jax.ShapeDtypeStruct((B, C, Do), jnp.float32),
        compiler_params=pltpu.CompilerParams(
            dimension_semantics=("parallel",)),
    )(A, pos_t, ctr, wr)


def _make_mlp_max_kernel(n_layers):
    def kern(*refs):
        x_ref = refs[0]
        o_ref = refs[-1]
        h = x_ref[0, :, :]
        for i in range(n_layers):
            w = refs[1 + 2 * i][...]
            b = refs[2 + 2 * i][...]
            h = jnp.tanh(jnp.dot(h, w, preferred_element_type=jnp.float32) + b)
        o_ref[0, :, :] = jnp.max(h, axis=0, keepdims=True)
    return kern


def _mlp_max(xp, params):
    B, M, D = xp.shape
    Do = params[-1][0].shape[1]
    in_specs = [pl.BlockSpec((1, M, D), lambda b: (b, 0, 0))]
    args = [xp]
    for (w, b) in params:
        dw_in, dw_out = w.shape
        in_specs.append(pl.BlockSpec((dw_in, dw_out), lambda b: (0, 0)))
        in_specs.append(pl.BlockSpec((1, dw_out), lambda b: (0, 0)))
        args.append(w)
        args.append(b.reshape(1, dw_out))
    out = pl.pallas_call(
        _make_mlp_max_kernel(len(params)),
        grid=(B,),
        in_specs=in_specs,
        out_specs=pl.BlockSpec((1, 1, Do), lambda b: (b, 0, 0)),
        out_shape=jax.ShapeDtypeStruct((B, 1, Do), jnp.float32),
        compiler_params=pltpu.CompilerParams(
            dimension_semantics=("parallel",)),
    )(*args)
    return out[:, 0, :]


def kernel(x, pos, p_sa1, p_sa2, p_b2, p_b3, p_b4):
    B, N, F = x.shape
    n1, n2, K = N // 2, N // 16, 64

    pos_t = jnp.transpose(pos, (0, 2, 1))
    xp = jnp.concatenate([x, pos], axis=-1)

    ctr1_t = _fps(pos_t, n1)
    ctr1 = jnp.transpose(ctr1_t, (0, 2, 1))

    w1a, b1a = p_sa1[0]
    w1b, b1b = p_b2[0]
    wcat = jnp.concatenate([w1a, w1b], axis=1)
    bcat = jnp.concatenate([b1a, b1b], axis=0)
    A1 = _affine(xp, wcat, bcat)
    wr_cat = wcat[F:F + 2, :]

    w2a_pad = jnp.zeros((128, 128), jnp.float32).at[:64, :].set(p_sa1[1][0])
    w2b_pad = jnp.zeros((128, 128), jnp.float32).at[64:, :].set(p_b2[1][0])
    x1a, x2 = _sa_dual(A1, pos_t, ctr1, wr_cat,
                       w2a_pad, p_sa1[1][1], w2b_pad, p_b2[1][1],
                       p_b2[2][0], p_b2[2][1], K, 0.6)

    ctr2_t = _fps(ctr1_t, n2)
    ctr2 = jnp.transpose(ctr2_t, (0, 2, 1))
    xp2 = jnp.concatenate([x1a, ctr1], axis=-1)
    w2, b2 = p_sa2[0]
    A2 = _affine(xp2, w2, b2)
    x1 = _sa_single(A2, ctr1_t, ctr2, w2[128:130, :], K, 0.8)

    x3 = _mlp_max(xp, p_b3)

    feat = jnp.concatenate([x1, x2], axis=1)
    posc = jnp.concatenate([ctr2, ctr1], axis=1)
    xp4 = jnp.concatenate([feat, posc], axis=-1)
    x4 = _mlp_max(xp4, p_b4)

    return jnp.concatenate([x3, x4], axis=-1)


# speedup vs baseline: 3.9163x; 1.0428x over previous
import functools

import jax
import jax.numpy as jnp
from jax.experimental import pallas as pl
from jax.experimental.pallas import tpu as pltpu


def _red2(op, a):
    return op(op(a, axis=1, keepdims=True), axis=0, keepdims=True)


def _fps_kernel(pos_ref, ctr_ref, *, n_pts, n_samples):
    S, L = pos_ref.shape[2], pos_ref.shape[3]
    CS, CL = ctr_ref.shape[2], ctr_ref.shape[3]
    px = pos_ref[0, 0, :, :]
    py = pos_ref[0, 1, :, :]
    pidx = (jax.lax.broadcasted_iota(jnp.int32, (S, L), 0) * L
            + jax.lax.broadcasted_iota(jnp.int32, (S, L), 1))
    cidx = (jax.lax.broadcasted_iota(jnp.int32, (CS, CL), 0) * CL
            + jax.lax.broadcasted_iota(jnp.int32, (CS, CL), 1))
    x0 = px[0:1, 0:1]
    y0 = py[0:1, 0:1]
    dmin0 = (px - x0) ** 2 + (py - y0) ** 2
    cx0 = jnp.where(cidx == 0, x0, 0.0)
    cy0 = jnp.where(cidx == 0, y0, 0.0)

    def body(i, carry):
        dmin, cxs, cys = carry
        m = _red2(jnp.max, dmin)
        cand = jnp.where(dmin == m, pidx, n_pts)
        jsel = _red2(jnp.min, cand)
        onehot = pidx == jsel
        nx = _red2(jnp.sum, jnp.where(onehot, px, 0.0))
        ny = _red2(jnp.sum, jnp.where(onehot, py, 0.0))
        dnew = (px - nx) ** 2 + (py - ny) ** 2
        return (jnp.minimum(dmin, dnew),
                jnp.where(cidx == i, nx, cxs),
                jnp.where(cidx == i, ny, cys))

    _, cxs, cys = jax.lax.fori_loop(1, n_samples, body, (dmin0, cx0, cy0))
    ctr_ref[0, 0, :, :] = cxs
    ctr_ref[0, 1, :, :] = cys


def _fps(pos_t, n_samples):
    B, _, N = pos_t.shape
    L = 256 if N % 256 == 0 else 128
    S = N // L
    CL = 256 if n_samples % 256 == 0 else 128
    CS = max(n_samples // CL, 1)
    CL = n_samples // CS
    pos_p = pos_t.reshape(B, 2, S, L)
    out = pl.pallas_call(
        functools.partial(_fps_kernel, n_pts=N, n_samples=n_samples),
        grid=(B,),
        in_specs=[pl.BlockSpec((1, 2, S, L), lambda b: (b, 0, 0, 0))],
        out_specs=pl.BlockSpec((1, 2, CS, CL), lambda b: (b, 0, 0, 0)),
        out_shape=jax.ShapeDtypeStruct((B, 2, CS, CL), jnp.float32),
        compiler_params=pltpu.CompilerParams(
            dimension_semantics=("parallel",)),
    )(pos_p)
    return out.reshape(B, 2, n_samples)


def _affine_kernel(x_ref, w_ref, b_ref, o_ref):
    o_ref[0, :, :] = (
        jnp.dot(x_ref[0, :, :], w_ref[...], preferred_element_type=jnp.float32)
        + b_ref[...]
    )


def _affine(xp, w, b):
    B, N, D = xp.shape
    Do = w.shape[1]
    return pl.pallas_call(
        _affine_kernel,
        grid=(B,),
        in_specs=[
            pl.BlockSpec((1, N, D), lambda bb: (bb, 0, 0)),
            pl.BlockSpec((D, Do), lambda bb: (0, 0)),
            pl.BlockSpec((1, Do), lambda bb: (0, 0)),
        ],
        out_specs=pl.BlockSpec((1, N, Do), lambda bb: (bb, 0, 0)),
        out_shape=jax.ShapeDtypeStruct((B, N, Do), jnp.float32),
        compiler_params=pltpu.CompilerParams(
            dimension_semantics=("parallel",)),
    )(xp, w, b.reshape(1, Do))


def _sa_dual_kernel(a_ref, pos_ref, ctr_ref, wr_ref, w2a_ref, b2a_ref,
                    w2b_ref, b2b_ref, w3b_ref, b3b_ref, oa_ref, ob_ref,
                    *, n_pts, tc, kk, r2):
    A = a_ref[0, :, :]
    A_hi = A.astype(jnp.bfloat16)
    A_lo = (A - A_hi.astype(jnp.float32)).astype(jnp.bfloat16)
    AHL = jnp.concatenate([A_hi, A_lo], axis=1)
    px = pos_ref[0, 0:1, :]
    py = pos_ref[0, 1:2, :]
    cx = ctr_ref[0, :, 0:1]
    cy = ctr_ref[0, :, 1:2]
    cp = jnp.dot(ctr_ref[0, :, :], pos_ref[0, :, :],
                 preferred_element_type=jnp.float32)
    d2 = (cx * cx + cy * cy) + (px * px + py * py) - 2.0 * cp
    neg0 = jnp.where(d2 <= r2, -d2, -jnp.inf)
    iota = jax.lax.broadcasted_iota(jnp.int32, (tc, n_pts), 1)
    ctrw = cx * wr_ref[0:1, :] + cy * wr_ref[1:2, :]
    acc_a0 = jnp.full((tc, 128), -jnp.inf, jnp.float32)
    acc_b0 = jnp.full((tc, 256), -jnp.inf, jnp.float32)

    def body(_, carry):
        neg, acc_a, acc_b = carry
        m = jnp.max(neg, axis=1, keepdims=True)
        valid = m > -jnp.inf
        cand = jnp.where(neg == m, iota, n_pts)
        jsel = jnp.min(cand, axis=1, keepdims=True)
        onehot = iota == jsel
        neg = jnp.where(onehot, -jnp.inf, neg)
        oh = onehot.astype(jnp.bfloat16)
        g2 = jnp.dot(oh, AHL, preferred_element_type=jnp.float32)
        g = g2[:, 0:128] + g2[:, 128:256]
        h1 = jnp.tanh(g - ctrw)
        h2a = jnp.tanh(jnp.dot(h1, w2a_ref[...],
                               preferred_element_type=jnp.float32)
                       + b2a_ref[...])
        acc_a = jnp.where(valid, jnp.maximum(acc_a, h2a), acc_a)
        h2b = jnp.tanh(jnp.dot(h1, w2b_ref[...],
                               preferred_element_type=jnp.float32)
                       + b2b_ref[...])
        h3b = jnp.tanh(jnp.dot(h2b, w3b_ref[...],
                               preferred_element_type=jnp.float32)
                       + b3b_ref[...])
        acc_b = jnp.where(valid, jnp.maximum(acc_b, h3b), acc_b)
        return neg, acc_a, acc_b

    _, acc_a, acc_b = jax.lax.fori_loop(0, kk, body, (neg0, acc_a0, acc_b0))
    oa_ref[0, :, :] = acc_a
    ob_ref[0, :, :] = acc_b


def _sa_dual(A, pos_t, ctr, wr, w2a, b2a, w2b, b2b, w3b, b3b, kk, radius):
    B, N, _ = A.shape
    C = ctr.shape[1]
    TC = 1024
    kern = functools.partial(_sa_dual_kernel, n_pts=N, tc=TC, kk=kk,
                             r2=radius * radius)
    return pl.pallas_call(
        kern,
        grid=(B, C // TC),
        in_specs=[
            pl.BlockSpec((1, N, 128), lambda b, t: (b, 0, 0)),
            pl.BlockSpec((1, 2, N), lambda b, t: (b, 0, 0)),
            pl.BlockSpec((1, TC, 2), lambda b, t: (b, t, 0)),
            pl.BlockSpec((2, 128), lambda b, t: (0, 0)),
            pl.BlockSpec((128, 128), lambda b, t: (0, 0)),
            pl.BlockSpec((1, 128), lambda b, t: (0, 0)),
            pl.BlockSpec((128, 128), lambda b, t: (0, 0)),
            pl.BlockSpec((1, 128), lambda b, t: (0, 0)),
            pl.BlockSpec((128, 256), lambda b, t: (0, 0)),
            pl.BlockSpec((1, 256), lambda b, t: (0, 0)),
        ],
        out_specs=[
            pl.BlockSpec((1, TC, 128), lambda b, t: (b, t, 0)),
            pl.BlockSpec((1, TC, 256), lambda b, t: (b, t, 0)),
        ],
        out_shape=[
            jax.ShapeDtypeStruct((B, C, 128), jnp.float32),
            jax.ShapeDtypeStruct((B, C, 256), jnp.float32),
        ],
        compiler_params=pltpu.CompilerParams(
            dimension_semantics=("parallel", "parallel")),
    )(A, pos_t, ctr, wr, w2a, b2a.reshape(1, -1), w2b, b2b.reshape(1, -1),
      w3b, b3b.reshape(1, -1))


def _sa_single_kernel(a_ref, pos_ref, ctr_ref, wr_ref, o_ref,
                      *, n_pts, tc, kk, r2, dout):
    A = a_ref[0, :, :]
    A_hi = A.astype(jnp.bfloat16)
    A_lo = (A - A_hi.astype(jnp.float32)).astype(jnp.bfloat16)
    AHL = jnp.concatenate([A_hi, A_lo], axis=1)
    px = pos_ref[0, 0:1, :]
    py = pos_ref[0, 1:2, :]
    cx = ctr_ref[0, :, 0:1]
    cy = ctr_ref[0, :, 1:2]
    cp = jnp.dot(ctr_ref[0, :, :], pos_ref[0, :, :],
                 preferred_element_type=jnp.float32)
    d2 = (cx * cx + cy * cy) + (px * px + py * py) - 2.0 * cp
    neg0 = jnp.where(d2 <= r2, -d2, -jnp.inf)
    iota = jax.lax.broadcasted_iota(jnp.int32, (tc, n_pts), 1)
    ctrw = cx * wr_ref[0:1, :] + cy * wr_ref[1:2, :]
    acc0 = jnp.full((tc, dout), -jnp.inf, jnp.float32)

    def body(_, carry):
        neg, acc = carry
        m = jnp.max(neg, axis=1, keepdims=True)
        valid = m > -jnp.inf
        cand = jnp.where(neg == m, iota, n_pts)
        jsel = jnp.min(cand, axis=1, keepdims=True)
        onehot = iota == jsel
        neg = jnp.where(onehot, -jnp.inf, neg)
        oh = onehot.astype(jnp.bfloat16)
        g2 = jnp.dot(oh, AHL, preferred_element_type=jnp.float32)
        g = g2[:, 0:dout] + g2[:, dout:2 * dout]
        h = jnp.tanh(g - ctrw)
        acc = jnp.where(valid, jnp.maximum(acc, h), acc)
        return neg, acc

    _, acc = jax.lax.fori_loop(0, kk, body, (neg0, acc0))
    o_ref[0, :, :] = acc


def _sa_single(A, pos_t, ctr, wr, kk, radius):
    B, N, Do = A.shape
    C = ctr.shape[1]
    kern = functools.partial(_sa_single_kernel, n_pts=N, tc=C, kk=kk,
                             r2=radius * radius, dout=Do)
    return pl.pallas_call(
        kern,
        grid=(B,),
        in_specs=[
            pl.BlockSpec((1, N, Do), lambda b: (b, 0, 0)),
            pl.BlockSpec((1, 2, N), lambda b: (b, 0, 0)),
            pl.BlockSpec((1, C, 2), lambda b: (b, 0, 0)),
            pl.BlockSpec((2, Do), lambda b: (0, 0)),
        ],
        out_specs=pl.BlockSpec((1, C, Do), lambda b: (b, 0, 0)),
        out_shape=jax.ShapeDtypeStruct((B, C, Do), jnp.float32),
        compiler_params=pltpu.CompilerParams(
            dimension_semantics=("parallel",)),
    )(A, pos_t, ctr, wr)


def _make_mlp_max_kernel(n_layers):
    def kern(*refs):
        x_ref = refs[0]
        o_ref = refs[-1]
        h = x_ref[0, :, :]
        for i in range(n_layers):
            w = refs[1 + 2 * i][...]
            b = refs[2 + 2 * i][...]
            h = jnp.tanh(jnp.dot(h, w, preferred_element_type=jnp.float32) + b)
        o_ref[0, :, :] = jnp.max(h, axis=0, keepdims=True)
    return kern


def _mlp_max(xp, params):
    B, M, D = xp.shape
    Do = params[-1][0].shape[1]
    in_specs = [pl.BlockSpec((1, M, D), lambda b: (b, 0, 0))]
    args = [xp]
    for (w, b) in params:
        dw_in, dw_out = w.shape
        in_specs.append(pl.BlockSpec((dw_in, dw_out), lambda b: (0, 0)))
        in_specs.append(pl.BlockSpec((1, dw_out), lambda b: (0, 0)))
        args.append(w)
        args.append(b.reshape(1, dw_out))
    out = pl.pallas_call(
        _make_mlp_max_kernel(len(params)),
        grid=(B,),
        in_specs=in_specs,
        out_specs=pl.BlockSpec((1, 1, Do), lambda b: (b, 0, 0)),
        out_shape=jax.ShapeDtypeStruct((B, 1, Do), jnp.float32),
        compiler_params=pltpu.CompilerParams(
            dimension_semantics=("parallel",)),
    )(*args)
    return out[:, 0, :]


def kernel(x, pos, p_sa1, p_sa2, p_b2, p_b3, p_b4):
    B, N, F = x.shape
    n1, n2, K = N // 2, N // 16, 64

    pos_t = jnp.transpose(pos, (0, 2, 1))
    xp = jnp.concatenate([x, pos], axis=-1)

    ctr1_t = _fps(pos_t, n1)
    ctr1 = jnp.transpose(ctr1_t, (0, 2, 1))

    w1a, b1a = p_sa1[0]
    w1b, b1b = p_b2[0]
    wcat = jnp.concatenate([w1a, w1b], axis=1)
    bcat = jnp.concatenate([b1a, b1b], axis=0)
    A1 = _affine(xp, wcat, bcat)
    wr_cat = wcat[F:F + 2, :]

    w2a_pad = jnp.zeros((128, 128), jnp.float32).at[:64, :].set(p_sa1[1][0])
    w2b_pad = jnp.zeros((128, 128), jnp.float32).at[64:, :].set(p_b2[1][0])
    x1a, x2 = _sa_dual(A1, pos_t, ctr1, wr_cat,
                       w2a_pad, p_sa1[1][1], w2b_pad, p_b2[1][1],
                       p_b2[2][0], p_b2[2][1], K, 0.6)

    ctr2_t = _fps(ctr1_t, n2)
    ctr2 = jnp.transpose(ctr2_t, (0, 2, 1))
    xp2 = jnp.concatenate([x1a, ctr1], axis=-1)
    w2, b2 = p_sa2[0]
    A2 = _affine(xp2, w2, b2)
    x1 = _sa_single(A2, ctr1_t, ctr2, w2[128:130, :], K, 0.8)

    x3 = _mlp_max(xp, p_b3)

    feat = jnp.concatenate([x1, x2], axis=1)
    posc = jnp.concatenate([ctr2, ctr1], axis=1)
    xp4 = jnp.concatenate([feat, posc], axis=-1)
    x4 = _mlp_max(xp4, p_b4)

    return jnp.concatenate([x3, x4], axis=-1)
